# baseline (speedup 1.0000x reference)
import functools

import jax
import jax.numpy as jnp
from jax import lax
from jax.experimental import pallas as pl
from jax.experimental.pallas import tpu as pltpu
from jax.experimental.pallas import tpu_sc as plsc

EMB = 64
CHUNK = 40
NCHUNK = 5


@functools.cache
def _make_gather(n_rows: int, row_len: int):
    info = plsc.get_sparse_core_info()
    nc, ns = info.num_cores, info.num_subcores
    nw = nc * ns
    per_w = n_rows // nw
    assert per_w % 2 == 0 and row_len == NCHUNK * CHUNK
    mesh = plsc.VectorSubcoreMesh(core_axis_name="c", subcore_axis_name="s")

    @functools.partial(
        pl.kernel,
        out_type=jax.ShapeDtypeStruct((n_rows, row_len, EMB), jnp.float32),
        mesh=mesh,
        scratch_types=[
            pltpu.VMEM((per_w, row_len), jnp.int32),
            pltpu.VMEM((NCHUNK, CHUNK, EMB), jnp.float32),
            pltpu.VMEM((NCHUNK, CHUNK, EMB), jnp.float32),
            pltpu.SemaphoreType.DMA,
            pltpu.SemaphoreType.DMA,
            pltpu.SemaphoreType.DMA,
            pltpu.SemaphoreType.DMA,
        ],
        compiler_params=pltpu.CompilerParams(use_tc_tiling_on_sc=False),
    )
    def gather_kernel(idx_hbm, table_hbm, out_hbm, idx_v, bufa, bufb,
                      gsema, gsemb, ssema, ssemb):
        wid = lax.axis_index("s") * nc + lax.axis_index("c")
        r0 = wid * per_w
        pltpu.sync_copy(idx_hbm.at[pl.ds(r0, per_w)], idx_v)

        def fire_gathers(arr, sem, g):
            for b in range(NCHUNK):
                pltpu.async_copy(
                    table_hbm.at[idx_v.at[g, pl.ds(b * CHUNK, CHUNK)]],
                    arr.at[b], sem)

        def fire_stores(arr, sem, g):
            for b in range(NCHUNK):
                pltpu.async_copy(
                    arr.at[b], out_hbm.at[r0 + g, pl.ds(b * CHUNK, CHUNK)], sem)

        def drain_gathers(arr, sem):
            for b in range(NCHUNK):
                pltpu.make_async_copy(
                    table_hbm.at[pl.ds(0, CHUNK)], arr.at[b], sem).wait()

        def drain_stores(arr, sem):
            for b in range(NCHUNK):
                pltpu.make_async_copy(
                    arr.at[b], out_hbm.at[0, pl.ds(b * CHUNK, CHUNK)], sem).wait()

        fire_gathers(bufa, gsema, 0)

        @pl.loop(0, per_w, step=2)
        def _(g):
            @pl.when(g > 0)
            def _():
                drain_stores(bufb, ssemb)
            fire_gathers(bufb, gsemb, g + 1)
            drain_gathers(bufa, gsema)
            fire_stores(bufa, ssema, g)
            drain_gathers(bufb, gsemb)
            fire_stores(bufb, ssemb, g + 1)

            @pl.when(g + 2 < per_w)
            def _():
                drain_stores(bufa, ssema)
                fire_gathers(bufa, gsema, g + 2)

        drain_stores(bufa, ssema)
        drain_stores(bufb, ssemb)

    return gather_kernel


def kernel(inputs, table):
    b, t = inputs.shape
    return _make_gather(b, t)(inputs.astype(jnp.int32), table)


# speedup vs baseline: 1.0095x; 1.0095x over previous
import functools

import jax
import jax.numpy as jnp
from jax import lax
from jax.experimental import pallas as pl
from jax.experimental.pallas import tpu as pltpu
from jax.experimental.pallas import tpu_sc as plsc

EMB = 64
BLK = 128
R = 4


@functools.cache
def _make_gather(n_b: int, n_t: int):
    info = plsc.get_sparse_core_info()
    nc, ns = info.num_cores, info.num_subcores
    nw = nc * ns
    assert n_b % (nw * BLK) == 0 and n_t % (2 * R) == 0
    blocks_per_w = n_b // (nw * BLK)
    mesh = plsc.VectorSubcoreMesh(core_axis_name="c", subcore_axis_name="s")

    @functools.partial(
        pl.kernel,
        out_type=jax.ShapeDtypeStruct((n_b, n_t, EMB), jnp.float32),
        mesh=mesh,
        scratch_types=[
            pltpu.VMEM((n_t, BLK), jnp.int32),
            pltpu.VMEM((R, BLK, EMB), jnp.float32),
            pltpu.VMEM((R, BLK, EMB), jnp.float32),
            pltpu.SemaphoreType.DMA,
            pltpu.SemaphoreType.DMA,
            pltpu.SemaphoreType.DMA,
            pltpu.SemaphoreType.DMA,
        ],
        compiler_params=pltpu.CompilerParams(use_tc_tiling_on_sc=False),
    )
    def gather_kernel(idxt_hbm, table_hbm, out_hbm, idx_v, bufa, bufb,
                      gsema, gsemb, ssema, ssemb):
        wid = lax.axis_index("s") * nc + lax.axis_index("c")
        b0 = wid * BLK
        pltpu.sync_copy(idxt_hbm.at[:, pl.ds(b0, BLK)], idx_v)

        def fire_gathers(arr, sem, t0):
            for r in range(R):
                pltpu.async_copy(
                    table_hbm.at[idx_v.at[t0 + r]], arr.at[r], sem)

        def fire_stores(arr, sem, t0):
            for r in range(R):
                pltpu.async_copy(
                    arr.at[r], out_hbm.at[pl.ds(b0, BLK), t0 + r], sem)

        def drain_gathers(arr, sem):
            for r in range(R):
                pltpu.make_async_copy(
                    table_hbm.at[pl.ds(0, BLK)], arr.at[r], sem).wait()

        def drain_stores(arr, sem):
            for r in range(R):
                pltpu.make_async_copy(
                    arr.at[r], out_hbm.at[pl.ds(0, BLK), 0], sem).wait()

        fire_gathers(bufa, gsema, 0)

        @pl.loop(0, n_t, step=2 * R)
        def _(t0):
            @pl.when(t0 > 0)
            def _():
                drain_stores(bufb, ssemb)
            fire_gathers(bufb, gsemb, t0 + R)
            drain_gathers(bufa, gsema)
            fire_stores(bufa, ssema, t0)
            drain_gathers(bufb, gsemb)
            fire_stores(bufb, ssemb, t0 + R)

            @pl.when(t0 + 2 * R < n_t)
            def _():
                drain_stores(bufa, ssema)
                fire_gathers(bufa, gsema, t0 + 2 * R)

        drain_stores(bufa, ssema)
        drain_stores(bufb, ssemb)

    del blocks_per_w
    return gather_kernel


def kernel(inputs, table):
    b, t = inputs.shape
    idxt = inputs.T.astype(jnp.int32)
    return _make_gather(b, t)(idxt, table)
